# baseline (speedup 1.0000x reference)
import functools

import jax
import jax.numpy as jnp
from jax.experimental import pallas as pl
from jax.experimental.pallas import tpu as pltpu

_H = 128


def _odefunc_body(g_ref, si_ref, bg_ref, wt_ref, b_ref, out_ref, k_ref,
                  *, block_rows):
    i = pl.program_id(0)
    B = block_rows

    @pl.when(i == 0)
    def _():
        lane = jax.lax.broadcasted_iota(jnp.int32, g_ref.shape, 1)
        is_g = (lane % 4) == 2
        k_ref[0] = jnp.sum(((g_ref[...] != 0.0) & is_g).astype(jnp.int32))

    k = k_ref[0]
    v = si_ref[...].reshape(2 * B, _H)
    sir = jax.nn.sigmoid(
        jax.lax.dot_general(
            v, wt_ref[...], (((1,), (0,)), ((), ())),
            preferred_element_type=jnp.float32,
        )
        + b_ref[...]
    )
    s = sir[0:B]
    ii = sir[B:2 * B]
    row = i * B + jax.lax.broadcasted_iota(jnp.int32, (B, 1), 0)
    mask = (row < k).astype(jnp.float32)
    beta = bg_ref[:, 0:1]
    gamma = bg_ref[:, 1:2]
    ds = -beta * (ii * mask * s)
    dr = gamma * ii
    out_ref[0] = ds
    out_ref[1] = -ds - dr
    out_ref[2] = dr
    out_ref[3] = jnp.zeros_like(ds)


def kernel(t, x, W, b):
    del t
    n = x.shape[1]
    block_rows = 5000
    bgg = x[3, :, 0:4]
    gflat = bgg.reshape(n // 40, 160)
    wt = W.T
    b2 = b.reshape(1, _H)
    out = pl.pallas_call(
        functools.partial(_odefunc_body, block_rows=block_rows),
        grid=(n // block_rows,),
        in_specs=[
            pl.BlockSpec((n // 40, 160), lambda i: (0, 0)),
            pl.BlockSpec((2, block_rows, _H), lambda i: (0, i, 0)),
            pl.BlockSpec((block_rows, 4), lambda i: (i, 0)),
            pl.BlockSpec((_H, _H), lambda i: (0, 0)),
            pl.BlockSpec((1, _H), lambda i: (0, 0)),
        ],
        out_specs=pl.BlockSpec((4, block_rows, _H), lambda i: (0, i, 0)),
        out_shape=jax.ShapeDtypeStruct((4, n, _H), jnp.float32),
        scratch_shapes=[pltpu.SMEM((1,), jnp.int32)],
    )(gflat, x, bgg, wt, b2)
    return out


# speedup vs baseline: 1.2999x; 1.2999x over previous
import functools

import jax
import jax.numpy as jnp
from jax.experimental import pallas as pl
from jax.experimental.pallas import tpu as pltpu

_H = 128
_GP_ROWS = 400


def _odefunc_body(g_ref, si_ref, x3_ref, wt_ref, b_ref, out_ref, k_ref,
                  *, block_rows):
    i = pl.program_id(0)
    B = block_rows

    @pl.when(i == 0)
    def _():
        k_ref[0] = jnp.sum((g_ref[...] != 0.0).astype(jnp.int32))

    k = k_ref[0]
    v = si_ref[...].reshape(2 * B, _H)
    sir = jax.nn.sigmoid(
        jax.lax.dot_general(
            v, wt_ref[...], (((1,), (0,)), ((), ())),
            preferred_element_type=jnp.float32,
        )
        + b_ref[...]
    )
    s = sir[0:B]
    ii = sir[B:2 * B]
    row = i * B + jax.lax.broadcasted_iota(jnp.int32, (B, 1), 0)
    mask = (row < k).astype(jnp.float32)
    beta = x3_ref[0, :, 0:1]
    gamma = x3_ref[0, :, 1:2]
    ds = -beta * (ii * mask * s)
    dr = gamma * ii
    out_ref[0] = ds
    out_ref[1] = -ds - dr
    out_ref[2] = dr
    out_ref[3] = jnp.zeros_like(ds)


def kernel(t, x, W, b):
    del t
    n = x.shape[1]
    block_rows = 5000
    gpad = jnp.pad(x[3, :, 2], (0, _GP_ROWS * 128 - n)).reshape(_GP_ROWS, 128)
    wt = W.T
    b2 = b.reshape(1, _H)
    out = pl.pallas_call(
        functools.partial(_odefunc_body, block_rows=block_rows),
        grid=(n // block_rows,),
        in_specs=[
            pl.BlockSpec((_GP_ROWS, 128), lambda i: (0, 0)),
            pl.BlockSpec((2, block_rows, _H), lambda i: (0, i, 0)),
            pl.BlockSpec((1, block_rows, _H), lambda i: (3, i, 0)),
            pl.BlockSpec((_H, _H), lambda i: (0, 0)),
            pl.BlockSpec((1, _H), lambda i: (0, 0)),
        ],
        out_specs=pl.BlockSpec((4, block_rows, _H), lambda i: (0, i, 0)),
        out_shape=jax.ShapeDtypeStruct((4, n, _H), jnp.float32),
        scratch_shapes=[pltpu.SMEM((1,), jnp.int32)],
    )(gpad, x, x, wt, b2)
    return out
